# baseline (speedup 1.0000x reference)
import functools

import jax
import jax.numpy as jnp
from jax import lax
from jax.experimental import pallas as pl
from jax.experimental.pallas import tpu as pltpu
from jax.experimental.pallas import tpu_sc as plsc

C = 256
OUT = 7
NPTS = OUT * OUT
NPTS2 = (NPTS + 1) // 2
NC = 2
NS = 16
NW = NC * NS
GRP = 16
FINEST = 56.0

_T1 = (FINEST * (2.0 - 1e-6)) ** 2
_T2 = (FINEST * (4.0 - 1e-6)) ** 2
_T3 = (FINEST * (8.0 - 1e-6)) ** 2


@functools.lru_cache(maxsize=None)
def _build_sc_kernel(npad, h0, off0, off1, off2, off3):
    per_w = npad // NW
    ngrp = per_w // GRP
    mesh = plsc.VectorSubcoreMesh(core_axis_name="c", subcore_axis_name="s")

    @functools.partial(
        pl.kernel,
        mesh=mesh,
        out_type=jax.ShapeDtypeStruct((NPTS * npad, C), jnp.float32),
        scratch_types=[
            pltpu.VMEM((5, per_w), jnp.float32),
            pltpu.VMEM((NPTS2 * 128,), jnp.int32),
            pltpu.VMEM((2 * NPTS2, 4, 32), jnp.float32),
            pltpu.VMEM((128, C // 2), jnp.int32),
            pltpu.VMEM((128, C // 2), jnp.int32),
            pltpu.VMEM((GRP, C), jnp.float32),
            pltpu.VMEM((GRP, C), jnp.float32),
            pltpu.SemaphoreType.DMA,
            pltpu.SemaphoreType.DMA,
            pltpu.SemaphoreType.DMA,
            pltpu.SemaphoreType.DMA,
        ],
    )
    def sc_kernel(table_h, rois_h, out_h, rois_v, idx_v, wts_v, rows0_v,
                  rows1_v, out0_v, out1_v, sem0, sem1, osem0, osem1):
        wid = lax.axis_index("s") * NC + lax.axis_index("c")
        base = wid * per_w
        pltpu.sync_copy(rois_h.at[wid], rois_v)

        def g_body(g, carry):
            s16 = pl.ds(g * GRP, GRP)
            bf = rois_v[0, s16]
            x1 = rois_v[1, s16]
            y1 = rois_v[2, s16]
            x2 = rois_v[3, s16]
            y2 = rois_v[4, s16]

            area = jnp.maximum((x2 - x1) * (y2 - y1), 1e-12)
            one_i = jnp.full((GRP,), 1, jnp.int32)
            zero_i = jnp.full((GRP,), 0, jnp.int32)
            lvl = (jnp.where(area >= _T1, one_i, zero_i)
                   + jnp.where(area >= _T2, one_i, zero_i)
                   + jnp.where(area >= _T3, one_i, zero_i))
            hi = jnp.right_shift(jnp.full((GRP,), h0, jnp.int32), lvl)
            hf = hi.astype(jnp.float32)
            inv = 1.0 / jnp.left_shift(jnp.full((GRP,), 4, jnp.int32),
                                       lvl).astype(jnp.float32)
            lvl_off = jnp.where(
                lvl == 0, jnp.full((GRP,), off0, jnp.int32),
                jnp.where(lvl == 1, jnp.full((GRP,), off1, jnp.int32),
                          jnp.where(lvl == 2, jnp.full((GRP,), off2, jnp.int32),
                                    jnp.full((GRP,), off3, jnp.int32))))
            cbase = lvl_off + bf.astype(jnp.int32) * (hi * hi)

            x1s = x1 * inv - 0.5
            y1s = y1 * inv - 0.5
            x2s = x2 * inv - 0.5
            y2s = y2 * inv - 0.5
            bw = (x2s - x1s) / float(OUT)
            bh = (y2s - y1s) / float(OUT)

            def p_body(p, carry2):
                pp = jnp.minimum(p, NPTS - 1)
                py = pp // OUT
                px = pp - py * OUT
                ox = jnp.broadcast_to(px.astype(jnp.float32) + 0.5, (GRP,))
                oy = jnp.broadcast_to(py.astype(jnp.float32) + 0.5, (GRP,))
                gx = x1s + bw * ox
                gy = y1s + bh * oy
                valid = ((gy > -1.0) & (gy < hf) & (gx > -1.0) & (gx < hf))
                yc = jnp.clip(gy, 0.0, hf - 1.0)
                xc = jnp.clip(gx, 0.0, hf - 1.0)
                y0 = jnp.minimum(yc.astype(jnp.int32), hi - 2)
                x0 = jnp.minimum(xc.astype(jnp.int32), hi - 2)
                ly = yc - y0.astype(jnp.float32)
                lx = xc - x0.astype(jnp.float32)
                hy = 1.0 - ly
                hx = 1.0 - lx
                vf = jnp.where(valid, jnp.full((GRP,), 1.0, jnp.float32),
                               jnp.full((GRP,), 0.0, jnp.float32))
                i00 = cbase + y0 * hi + x0
                idx_v[pl.ds(p * 64, 16)] = i00
                idx_v[pl.ds(p * 64 + 16, 16)] = i00 + 1
                idx_v[pl.ds(p * 64 + 32, 16)] = i00 + hi
                idx_v[pl.ds(p * 64 + 48, 16)] = i00 + hi + 1
                wts_v[p, 0, pl.ds(0, 16)] = (hy * hx) * vf
                wts_v[p, 1, pl.ds(0, 16)] = (hy * lx) * vf
                wts_v[p, 2, pl.ds(0, 16)] = (ly * hx) * vf
                wts_v[p, 3, pl.ds(0, 16)] = (ly * lx) * vf
                return carry2
            lax.fori_loop(0, 2 * NPTS2, p_body, 0)

            def fire(q, rows_ref, sem):
                return pltpu.async_copy(
                    table_h.at[idx_v.at[pl.ds(q * 128, 128)]], rows_ref, sem)

            def drain(q, rows_ref, sem):
                pltpu.make_async_copy(
                    table_h.at[idx_v.at[pl.ds(q * 128, 128)]], rows_ref,
                    sem).wait()

            def out_slot(p):
                return out_h.at[pl.ds(p * npad + base + g * GRP, GRP), :]

            def combine(p, rows_ref, off, out_ref, osem):
                @pl.when(p >= 2)
                def _():
                    pltpu.make_async_copy(out_ref, out_slot(p - 2),
                                          osem).wait()

                def r_body(r, carry3):
                    s00 = wts_v[p, 0, pl.ds(r, 16)][0]
                    s01 = wts_v[p, 1, pl.ds(r, 16)][0]
                    s10 = wts_v[p, 2, pl.ds(r, 16)][0]
                    s11 = wts_v[p, 3, pl.ds(r, 16)][0]
                    def _split(w):
                        a = lax.bitcast_convert_type(
                            w << jnp.full((16,), 16, jnp.int32), jnp.float32)
                        b = lax.bitcast_convert_type(
                            w & jnp.full((16,), -65536, jnp.int32),
                            jnp.float32)
                        return a, b

                    for cb in range(C // 32):
                        s = pl.ds(cb * 16, 16)
                        a00, b00 = _split(rows_ref[off + r, s])
                        a01, b01 = _split(rows_ref[off + 16 + r, s])
                        a10, b10 = _split(rows_ref[off + 32 + r, s])
                        a11, b11 = _split(rows_ref[off + 48 + r, s])
                        out_ref[r, pl.ds(cb * 32, 16)] = (
                            a00 * s00 + a01 * s01 + a10 * s10 + a11 * s11)
                        out_ref[r, pl.ds(cb * 32 + 16, 16)] = (
                            b00 * s00 + b01 * s01 + b10 * s10 + b11 * s11)
                    return carry3
                lax.fori_loop(0, GRP, r_body, 0)
                pltpu.async_copy(out_ref, out_slot(p), osem)

            def combine2(q, rows_ref):
                combine(2 * q, rows_ref, 0, out0_v, osem0)

                @pl.when(2 * q + 1 < NPTS)
                def _():
                    combine(2 * q + 1, rows_ref, 64, out1_v, osem1)

            fire(0, rows0_v, sem0)

            def q_body(q, carry2):
                even = (q % 2) == 0

                @pl.when(jnp.logical_and(even, q + 1 < NPTS2))
                def _():
                    fire(q + 1, rows1_v, sem1)

                @pl.when(jnp.logical_and(jnp.logical_not(even), q + 1 < NPTS2))
                def _():
                    fire(q + 1, rows0_v, sem0)

                @pl.when(even)
                def _():
                    drain(q, rows0_v, sem0)
                    combine2(q, rows0_v)

                @pl.when(jnp.logical_not(even))
                def _():
                    drain(q, rows1_v, sem1)
                    combine2(q, rows1_v)
                return carry2
            lax.fori_loop(0, NPTS2, q_body, 0)
            pltpu.make_async_copy(out1_v, out_slot(NPTS - 2), osem1).wait()
            pltpu.make_async_copy(out0_v, out_slot(NPTS - 1), osem0).wait()
            return carry
        lax.fori_loop(0, ngrp, g_body, 0)

    return sc_kernel


def kernel(feat0, feat1, feat2, feat3, rois):
    feats = (feat0, feat1, feat2, feat3)
    tables = []
    offs = []
    row = 0
    for f in feats:
        b, c, h, w = f.shape
        offs.append(row)
        row += b * h * w
        tables.append(jnp.transpose(f, (0, 2, 3, 1)).reshape(b * h * w, c))
    table = jnp.concatenate(tables, axis=0).astype(jnp.bfloat16)
    table = jax.lax.bitcast_convert_type(
        table.reshape(-1, C // 2, 2), jnp.int32)

    n = rois.shape[0]
    npad = ((n + (NW * GRP) - 1) // (NW * GRP)) * (NW * GRP)
    rois_t = jnp.zeros((5, npad), jnp.float32).at[:, :n].set(rois.T)
    rois_t = rois_t.reshape(5, NW, npad // NW).transpose(1, 0, 2)

    h0 = feat0.shape[2]
    sc = _build_sc_kernel(npad, h0, offs[0], offs[1], offs[2], offs[3])
    out = sc(table, rois_t)
    import numpy as _np
    stored_true = _np.empty(C, _np.int32)
    for cb in range(C // 32):
        for k in range(16):
            stored_true[cb * 32 + k] = cb * 32 + 2 * k
            stored_true[cb * 32 + 16 + k] = cb * 32 + 2 * k + 1
    inv = _np.argsort(stored_true)
    out = out.reshape(NPTS, npad, C)[:, :n]
    out = out[:, :, jnp.asarray(inv)]
    return out.transpose(1, 2, 0).reshape(n, C, OUT, OUT)


# speedup vs baseline: 2.5448x; 2.5448x over previous
import functools

import jax
import jax.numpy as jnp
from jax import lax
from jax.experimental import pallas as pl
from jax.experimental.pallas import tpu as pltpu
from jax.experimental.pallas import tpu_sc as plsc

C = 256
OUT = 7
NPTS = OUT * OUT
NPTS2 = (NPTS + 1) // 2
NC = 2
NS = 16
NW = NC * NS
GRP = 16
FINEST = 56.0

_T1 = (FINEST * (2.0 - 1e-6)) ** 2
_T2 = (FINEST * (4.0 - 1e-6)) ** 2
_T3 = (FINEST * (8.0 - 1e-6)) ** 2


@functools.lru_cache(maxsize=None)
def _build_sc_kernel(npad, h0, off0, off1, off2, off3):
    per_w = npad // NW
    ngrp = per_w // GRP
    mesh = plsc.VectorSubcoreMesh(core_axis_name="c", subcore_axis_name="s")

    @functools.partial(
        pl.kernel,
        mesh=mesh,
        out_type=jax.ShapeDtypeStruct((NPTS * npad, C), jnp.float32),
        scratch_types=[
            pltpu.VMEM((5, per_w), jnp.float32),
            pltpu.VMEM((NPTS2 * 128,), jnp.int32),
            pltpu.VMEM((2 * NPTS2, 4, 32), jnp.float32),
            pltpu.VMEM((64, C), jnp.float32),
            pltpu.VMEM((64, C), jnp.float32),
            pltpu.VMEM((64, C), jnp.float32),
            pltpu.VMEM((64, C), jnp.float32),
            pltpu.VMEM((GRP, C), jnp.float32),
            pltpu.VMEM((GRP, C), jnp.float32),
            pltpu.SemaphoreType.DMA,
            pltpu.SemaphoreType.DMA,
            pltpu.SemaphoreType.DMA,
            pltpu.SemaphoreType.DMA,
            pltpu.SemaphoreType.DMA,
            pltpu.SemaphoreType.DMA,
        ],
    )
    def sc_kernel(table_h, rois_h, out_h, rois_v, idx_v, wts_v, rows0_v,
                  rows1_v, rows2_v, rows3_v, out0_v, out1_v, sem0, sem1,
                  sem2, sem3, osem0, osem1):
        wid = lax.axis_index("s") * NC + lax.axis_index("c")
        base = wid * per_w
        pltpu.sync_copy(rois_h.at[wid], rois_v)

        def g_body(g, carry):
            s16 = pl.ds(g * GRP, GRP)
            bf = rois_v[0, s16]
            x1 = rois_v[1, s16]
            y1 = rois_v[2, s16]
            x2 = rois_v[3, s16]
            y2 = rois_v[4, s16]

            area = jnp.maximum((x2 - x1) * (y2 - y1), 1e-12)
            one_i = jnp.full((GRP,), 1, jnp.int32)
            zero_i = jnp.full((GRP,), 0, jnp.int32)
            lvl = (jnp.where(area >= _T1, one_i, zero_i)
                   + jnp.where(area >= _T2, one_i, zero_i)
                   + jnp.where(area >= _T3, one_i, zero_i))
            hi = jnp.right_shift(jnp.full((GRP,), h0, jnp.int32), lvl)
            hf = hi.astype(jnp.float32)
            inv = 1.0 / jnp.left_shift(jnp.full((GRP,), 4, jnp.int32),
                                       lvl).astype(jnp.float32)
            lvl_off = jnp.where(
                lvl == 0, jnp.full((GRP,), off0, jnp.int32),
                jnp.where(lvl == 1, jnp.full((GRP,), off1, jnp.int32),
                          jnp.where(lvl == 2, jnp.full((GRP,), off2, jnp.int32),
                                    jnp.full((GRP,), off3, jnp.int32))))
            cbase = lvl_off + bf.astype(jnp.int32) * (hi * hi)

            x1s = x1 * inv - 0.5
            y1s = y1 * inv - 0.5
            x2s = x2 * inv - 0.5
            y2s = y2 * inv - 0.5
            bw = (x2s - x1s) / float(OUT)
            bh = (y2s - y1s) / float(OUT)

            def p_body(p, carry2):
                pp = jnp.minimum(p, NPTS - 1)
                py = pp // OUT
                px = pp - py * OUT
                ox = jnp.broadcast_to(px.astype(jnp.float32) + 0.5, (GRP,))
                oy = jnp.broadcast_to(py.astype(jnp.float32) + 0.5, (GRP,))
                gx = x1s + bw * ox
                gy = y1s + bh * oy
                valid = ((gy > -1.0) & (gy < hf) & (gx > -1.0) & (gx < hf))
                yc = jnp.clip(gy, 0.0, hf - 1.0)
                xc = jnp.clip(gx, 0.0, hf - 1.0)
                y0 = jnp.minimum(yc.astype(jnp.int32), hi - 2)
                x0 = jnp.minimum(xc.astype(jnp.int32), hi - 2)
                ly = yc - y0.astype(jnp.float32)
                lx = xc - x0.astype(jnp.float32)
                hy = 1.0 - ly
                hx = 1.0 - lx
                vf = jnp.where(valid, jnp.full((GRP,), 1.0, jnp.float32),
                               jnp.full((GRP,), 0.0, jnp.float32))
                i00 = cbase + y0 * hi + x0
                idx_v[pl.ds(p * 64, 16)] = i00
                idx_v[pl.ds(p * 64 + 16, 16)] = i00 + 1
                idx_v[pl.ds(p * 64 + 32, 16)] = i00 + hi
                idx_v[pl.ds(p * 64 + 48, 16)] = i00 + hi + 1
                wts_v[p, 0, pl.ds(0, 16)] = (hy * hx) * vf
                wts_v[p, 1, pl.ds(0, 16)] = (hy * lx) * vf
                wts_v[p, 2, pl.ds(0, 16)] = (ly * hx) * vf
                wts_v[p, 3, pl.ds(0, 16)] = (ly * lx) * vf
                return carry2
            lax.fori_loop(0, 2 * NPTS2, p_body, 0)

            def fire(p, rows_ref, sem):
                return pltpu.async_copy(
                    table_h.at[idx_v.at[pl.ds(p * 64, 64)]], rows_ref, sem)

            def drain(p, rows_ref, sem):
                pltpu.make_async_copy(
                    table_h.at[idx_v.at[pl.ds(p * 64, 64)]], rows_ref,
                    sem).wait()

            def out_slot(p):
                return out_h.at[pl.ds(p * npad + base + g * GRP, GRP), :]

            def combine(p, rows_ref, off, out_ref, osem):
                @pl.when(p >= 2)
                def _():
                    pltpu.make_async_copy(out_ref, out_slot(p - 2),
                                          osem).wait()

                def r_body(r, carry3):
                    s00 = wts_v[p, 0, pl.ds(r, 16)][0]
                    s01 = wts_v[p, 1, pl.ds(r, 16)][0]
                    s10 = wts_v[p, 2, pl.ds(r, 16)][0]
                    s11 = wts_v[p, 3, pl.ds(r, 16)][0]
                    for cb in range(C // 16):
                        s = pl.ds(cb * 16, 16)
                        out_ref[r, s] = (rows_ref[off + r, s] * s00
                                         + rows_ref[off + 16 + r, s] * s01
                                         + rows_ref[off + 32 + r, s] * s10
                                         + rows_ref[off + 48 + r, s] * s11)
                    return carry3
                lax.fori_loop(0, GRP, r_body, 0)
                pltpu.async_copy(out_ref, out_slot(p), osem)

            bufs = ((rows0_v, sem0), (rows1_v, sem1), (rows2_v, sem2),
                    (rows3_v, sem3))
            outs = ((out0_v, osem0), (out1_v, osem1))
            fire(0, rows0_v, sem0)
            fire(1, rows1_v, sem1)
            fire(2, rows2_v, sem2)

            def q_body(q, carry2):
                qm = q % 4
                for m in range(4):
                    rows_m, sem_m = bufs[m]
                    rows_n, sem_n = bufs[(m + 3) % 4]
                    out_m, osem_m = outs[m % 2]

                    @pl.when(jnp.logical_and(qm == m, q + 3 < NPTS))
                    def _(rows_n=rows_n, sem_n=sem_n):
                        fire(q + 3, rows_n, sem_n)

                    @pl.when(qm == m)
                    def _(rows_m=rows_m, sem_m=sem_m, out_m=out_m,
                          osem_m=osem_m):
                        drain(q, rows_m, sem_m)
                        combine(q, rows_m, 0, out_m, osem_m)
                return carry2
            lax.fori_loop(0, NPTS, q_body, 0)
            pltpu.make_async_copy(out1_v, out_slot(NPTS - 2), osem1).wait()
            pltpu.make_async_copy(out0_v, out_slot(NPTS - 1), osem0).wait()
            return carry
        lax.fori_loop(0, ngrp, g_body, 0)

    return sc_kernel


def kernel(feat0, feat1, feat2, feat3, rois):
    feats = (feat0, feat1, feat2, feat3)
    tables = []
    offs = []
    row = 0
    for f in feats:
        b, c, h, w = f.shape
        offs.append(row)
        row += b * h * w
        tables.append(jnp.transpose(f, (0, 2, 3, 1)).reshape(b * h * w, c))
    table = jnp.concatenate(tables, axis=0)

    n = rois.shape[0]
    npad = ((n + (NW * GRP) - 1) // (NW * GRP)) * (NW * GRP)
    rois_t = jnp.zeros((5, npad), jnp.float32).at[:, :n].set(rois.T)
    rois_t = rois_t.reshape(5, NW, npad // NW).transpose(1, 0, 2)

    h0 = feat0.shape[2]
    sc = _build_sc_kernel(npad, h0, offs[0], offs[1], offs[2], offs[3])
    out = sc(table, rois_t)
    out = out.reshape(NPTS, npad, C)[:, :n]
    return out.transpose(1, 2, 0).reshape(n, C, OUT, OUT)
